# baseline (speedup 1.0000x reference)
import functools

import jax
import jax.numpy as jnp
from jax import lax
from jax.experimental import pallas as pl
from jax.experimental.pallas import tpu as pltpu
from jax.experimental.pallas import tpu_sc as plsc

N = 10000
E = 320000
D = 128

NC = 2
NS = 16
NW = NC * NS
EPW = E // NW
CHUNK = 64
NCHUNK = 160
EPAD = NW * NCHUNK * CHUNK
NPAD = 10112
RPT = NPAD // NS

@functools.cache
def _mesh():
    return plsc.VectorSubcoreMesh(
        core_axis_name="c", subcore_axis_name="s", num_cores=NC, num_subcores=NS
    )


def _deg_body(dst2d, zeros_hbm, out, idx_v, ones_v, acc):
    c = lax.axis_index("c")
    s = lax.axis_index("s")
    wid = c * NS + s
    pltpu.sync_copy(dst2d.at[wid], idx_v)
    ones = jnp.ones((16,), jnp.float32)

    def obody(i, carry):
        r = i // (D // 16)
        k = i % (D // 16)
        ones_v[r, pl.ds(k * 16, 16)] = ones
        return carry

    lax.fori_loop(0, CHUNK * (D // 16), obody, 0)
    pltpu.sync_copy(zeros_hbm.at[pl.ds(s * RPT, RPT)], acc.at[pl.ds(s * RPT, RPT)])
    plsc.subcore_barrier()

    def body(j, carry):
        pltpu.sync_copy(ones_v, acc.at[idx_v.at[j]], add=True)
        return carry

    lax.fori_loop(0, NCHUNK, body, 0)
    plsc.subcore_barrier()
    pltpu.sync_copy(acc.at[pl.ds(s * RPT, RPT)], out.at[c, pl.ds(s * RPT, RPT)])


@functools.cache
def _deg_kernel():
    return pl.kernel(
        _deg_body,
        out_type=jax.ShapeDtypeStruct((NC, NPAD, D), jnp.float32),
        mesh=_mesh(),
        scratch_types=[
            pltpu.VMEM((NCHUNK, CHUNK), jnp.int32),
            pltpu.VMEM((CHUNK, D), jnp.float32),
            pltpu.VMEM_SHARED((NPAD, D), jnp.float32),
        ],
    )


def _agg_body(y, src2d, dst2d, zeros_hbm, out, sidx_v, didx_v, rows_v, sem, acc):
    c = lax.axis_index("c")
    s = lax.axis_index("s")
    wid = c * NS + s
    pltpu.sync_copy(src2d.at[wid], sidx_v)
    pltpu.sync_copy(dst2d.at[wid], didx_v)
    pltpu.sync_copy(zeros_hbm.at[pl.ds(s * RPT, RPT)], acc.at[pl.ds(s * RPT, RPT)])
    plsc.subcore_barrier()

    def body(j, carry):
        pltpu.async_copy(y.at[sidx_v.at[j]], rows_v, sem).wait()
        pltpu.sync_copy(rows_v, acc.at[didx_v.at[j]], add=True)
        return carry

    lax.fori_loop(0, NCHUNK, body, 0)
    plsc.subcore_barrier()
    pltpu.sync_copy(acc.at[pl.ds(s * RPT, RPT)], out.at[c, pl.ds(s * RPT, RPT)])


@functools.cache
def _agg_kernel():
    return pl.kernel(
        _agg_body,
        out_type=jax.ShapeDtypeStruct((NC, NPAD, D), jnp.float32),
        mesh=_mesh(),
        scratch_types=[
            pltpu.VMEM((NCHUNK, CHUNK), jnp.int32),
            pltpu.VMEM((NCHUNK, CHUNK), jnp.int32),
            pltpu.VMEM((CHUNK, D), jnp.float32),
            pltpu.SemaphoreType.DMA,
            pltpu.VMEM_SHARED((NPAD, D), jnp.float32),
        ],
    )


_BR = 2528
_GRID = NPAD // _BR


def _matmul_body(x_ref, w_ref, xw_ref):
    xw_ref[...] = jnp.dot(x_ref[...], w_ref[...],
                          preferred_element_type=jnp.float32)


def _matmul(x, W):
    return pl.pallas_call(
        _matmul_body,
        grid=(_GRID,),
        in_specs=[
            pl.BlockSpec((_BR, D), lambda i: (i, 0)),
            pl.BlockSpec((D, D), lambda i: (0, 0)),
        ],
        out_specs=pl.BlockSpec((_BR, D), lambda i: (i, 0)),
        out_shape=jax.ShapeDtypeStruct((NPAD, D), jnp.float32),
    )(x, W)


def _scale_body(xw_ref, degp_ref, y_ref, dis_ref):
    deg = degp_ref[0, :, :1] + degp_ref[1, :, :1] + 1.0
    dis = lax.rsqrt(deg)
    y_ref[...] = xw_ref[...] * dis
    dis_ref[...] = jnp.broadcast_to(dis, (_BR, 16))


def _first_layer(xw, degp):
    return pl.pallas_call(
        _scale_body,
        grid=(_GRID,),
        in_specs=[
            pl.BlockSpec((_BR, D), lambda i: (i, 0)),
            pl.BlockSpec((NC, _BR, D), lambda i: (0, i, 0)),
        ],
        out_specs=[
            pl.BlockSpec((_BR, D), lambda i: (i, 0)),
            pl.BlockSpec((_BR, 16), lambda i: (i, 0)),
        ],
        out_shape=[
            jax.ShapeDtypeStruct((NPAD, D), jnp.float32),
            jax.ShapeDtypeStruct((NPAD, 16), jnp.float32),
        ],
    )(xw, degp)


def _mid_body(p_ref, y1_ref, dis_ref, b1_ref, w2_ref, y2_ref):
    agg = p_ref[0] + p_ref[1] + y1_ref[...]
    dcol = dis_ref[:, :1]
    h = jnp.maximum(agg * dcol + b1_ref[...], 0.0)
    y2_ref[...] = jnp.dot(h, w2_ref[...], preferred_element_type=jnp.float32) * dcol


def _mid_layer(p, y1, dis, b1, W2):
    return pl.pallas_call(
        _mid_body,
        grid=(_GRID,),
        in_specs=[
            pl.BlockSpec((NC, _BR, D), lambda i: (0, i, 0)),
            pl.BlockSpec((_BR, D), lambda i: (i, 0)),
            pl.BlockSpec((_BR, 16), lambda i: (i, 0)),
            pl.BlockSpec((1, D), lambda i: (0, 0)),
            pl.BlockSpec((D, D), lambda i: (0, 0)),
        ],
        out_specs=pl.BlockSpec((_BR, D), lambda i: (i, 0)),
        out_shape=jax.ShapeDtypeStruct((NPAD, D), jnp.float32),
    )(p, y1, dis, b1, W2)


def _final_body(q_ref, y2_ref, dis_ref, b2_ref, out_ref):
    agg = q_ref[0] + q_ref[1] + y2_ref[...]
    out_ref[...] = jnp.maximum(agg * dis_ref[:, :1] + b2_ref[...], 0.0)


def _final_layer(q, y2, dis, b2):
    return pl.pallas_call(
        _final_body,
        grid=(_GRID,),
        in_specs=[
            pl.BlockSpec((NC, _BR, D), lambda i: (0, i, 0)),
            pl.BlockSpec((_BR, D), lambda i: (i, 0)),
            pl.BlockSpec((_BR, 16), lambda i: (i, 0)),
            pl.BlockSpec((1, D), lambda i: (0, 0)),
        ],
        out_specs=pl.BlockSpec((_BR, D), lambda i: (i, 0)),
        out_shape=jax.ShapeDtypeStruct((NPAD, D), jnp.float32),
    )(q, y2, dis, b2)


def kernel(x, edge_index, W1, b1, W2, b2):
    npad_e = EPAD - E
    src2d = jnp.concatenate(
        [edge_index[0].astype(jnp.int32), jnp.zeros((npad_e,), jnp.int32)]
    ).reshape(NW, NCHUNK, CHUNK)
    dst2d = jnp.concatenate(
        [edge_index[1].astype(jnp.int32), jnp.full((npad_e,), NPAD - 1, jnp.int32)]
    ).reshape(NW, NCHUNK, CHUNK)
    xp = jnp.pad(x, ((0, NPAD - N), (0, 0)))
    zeros128 = jnp.zeros((NPAD, D), jnp.float32)
    b1r = b1.reshape(1, D)
    b2r = b2.reshape(1, D)

    degp = _deg_kernel()(dst2d, zeros128)
    xw1 = _matmul(xp, W1)
    y1, dis = _first_layer(xw1, degp)
    p = _agg_kernel()(y1, src2d, dst2d, zeros128)
    y2 = _mid_layer(p, y1, dis, b1r, W2)
    q = _agg_kernel()(y2, src2d, dst2d, zeros128)
    return _final_layer(q, y2, dis, b2r)[:N]


# speedup vs baseline: 2.5058x; 2.5058x over previous
import functools

import jax
import jax.numpy as jnp
from jax import lax
from jax.experimental import pallas as pl
from jax.experimental.pallas import tpu as pltpu
from jax.experimental.pallas import tpu_sc as plsc

N = 10000
E = 320000
D = 128

NC = 2
NS = 16
NW = NC * NS
EPW = E // NW
CHUNK = 128
NCHUNK = 80
EPAD = NW * NCHUNK * CHUNK
NPAD = 10112
RPT = NPAD // NS

@functools.cache
def _mesh():
    return plsc.VectorSubcoreMesh(
        core_axis_name="c", subcore_axis_name="s", num_cores=NC, num_subcores=NS
    )


def _deg_body(dst2d, zeros_hbm, out, idx_v, ones_v, acc):
    c = lax.axis_index("c")
    s = lax.axis_index("s")
    wid = c * NS + s
    pltpu.sync_copy(dst2d.at[wid], idx_v)
    ones = jnp.ones((16,), jnp.float32)

    def obody(i, carry):
        r = i // (D // 16)
        k = i % (D // 16)
        ones_v[r, pl.ds(k * 16, 16)] = ones
        return carry

    lax.fori_loop(0, CHUNK * (D // 16), obody, 0)
    pltpu.sync_copy(zeros_hbm.at[pl.ds(s * RPT, RPT)], acc.at[pl.ds(s * RPT, RPT)])
    plsc.subcore_barrier()

    def body(j, carry):
        pltpu.sync_copy(ones_v, acc.at[idx_v.at[j]], add=True)
        return carry

    lax.fori_loop(0, NCHUNK, body, 0)
    plsc.subcore_barrier()
    pltpu.sync_copy(acc.at[pl.ds(s * RPT, RPT)], out.at[c, pl.ds(s * RPT, RPT)])


@functools.cache
def _deg_kernel():
    return pl.kernel(
        _deg_body,
        out_type=jax.ShapeDtypeStruct((NC, NPAD, D), jnp.float32),
        mesh=_mesh(),
        scratch_types=[
            pltpu.VMEM((NCHUNK, CHUNK), jnp.int32),
            pltpu.VMEM((CHUNK, D), jnp.float32),
            pltpu.VMEM_SHARED((NPAD, D), jnp.float32),
        ],
    )


def _agg_body(y, src2d, dst2d, zeros_hbm, out, sidx_v, didx_v, rows_v, sem, acc):
    c = lax.axis_index("c")
    s = lax.axis_index("s")
    wid = c * NS + s
    pltpu.sync_copy(src2d.at[wid], sidx_v)
    pltpu.sync_copy(dst2d.at[wid], didx_v)
    pltpu.sync_copy(zeros_hbm.at[pl.ds(s * RPT, RPT)], acc.at[pl.ds(s * RPT, RPT)])
    plsc.subcore_barrier()

    def body(j, carry):
        pltpu.async_copy(y.at[sidx_v.at[j]], rows_v, sem).wait()
        pltpu.sync_copy(rows_v, acc.at[didx_v.at[j]], add=True)
        return carry

    lax.fori_loop(0, NCHUNK, body, 0)
    plsc.subcore_barrier()
    pltpu.sync_copy(acc.at[pl.ds(s * RPT, RPT)], out.at[c, pl.ds(s * RPT, RPT)])


@functools.cache
def _agg_kernel():
    return pl.kernel(
        _agg_body,
        out_type=jax.ShapeDtypeStruct((NC, NPAD, D), jnp.float32),
        mesh=_mesh(),
        scratch_types=[
            pltpu.VMEM((NCHUNK, CHUNK), jnp.int32),
            pltpu.VMEM((NCHUNK, CHUNK), jnp.int32),
            pltpu.VMEM((CHUNK, D), jnp.float32),
            pltpu.SemaphoreType.DMA,
            pltpu.VMEM_SHARED((NPAD, D), jnp.float32),
        ],
    )


_BR = 2528
_GRID = NPAD // _BR


def _matmul_body(x_ref, w_ref, xw_ref):
    xw_ref[...] = jnp.dot(x_ref[...], w_ref[...],
                          preferred_element_type=jnp.float32)


def _matmul(x, W):
    return pl.pallas_call(
        _matmul_body,
        grid=(_GRID,),
        in_specs=[
            pl.BlockSpec((_BR, D), lambda i: (i, 0)),
            pl.BlockSpec((D, D), lambda i: (0, 0)),
        ],
        out_specs=pl.BlockSpec((_BR, D), lambda i: (i, 0)),
        out_shape=jax.ShapeDtypeStruct((NPAD, D), jnp.float32),
    )(x, W)


def _scale_body(xw_ref, degp_ref, y_ref, dis_ref):
    deg = degp_ref[0, :, :1] + degp_ref[1, :, :1] + 1.0
    dis = lax.rsqrt(deg)
    y_ref[...] = xw_ref[...] * dis
    dis_ref[...] = jnp.broadcast_to(dis, (_BR, 16))


def _first_layer(xw, degp):
    return pl.pallas_call(
        _scale_body,
        grid=(_GRID,),
        in_specs=[
            pl.BlockSpec((_BR, D), lambda i: (i, 0)),
            pl.BlockSpec((NC, _BR, D), lambda i: (0, i, 0)),
        ],
        out_specs=[
            pl.BlockSpec((_BR, D), lambda i: (i, 0)),
            pl.BlockSpec((_BR, 16), lambda i: (i, 0)),
        ],
        out_shape=[
            jax.ShapeDtypeStruct((NPAD, D), jnp.float32),
            jax.ShapeDtypeStruct((NPAD, 16), jnp.float32),
        ],
    )(xw, degp)


def _mid_body(p_ref, y1_ref, dis_ref, b1_ref, w2_ref, y2_ref):
    agg = p_ref[0] + p_ref[1] + y1_ref[...]
    dcol = dis_ref[:, :1]
    h = jnp.maximum(agg * dcol + b1_ref[...], 0.0)
    y2 = jnp.dot(h, w2_ref[...], preferred_element_type=jnp.float32) * dcol
    row = pl.program_id(0) * _BR + lax.broadcasted_iota(jnp.int32, (_BR, 1), 0)
    y2_ref[...] = jnp.where(row < N, y2, 0.0)


def _mid_layer(p, y1, dis, b1, W2):
    return pl.pallas_call(
        _mid_body,
        grid=(_GRID,),
        in_specs=[
            pl.BlockSpec((NC, _BR, D), lambda i: (0, i, 0)),
            pl.BlockSpec((_BR, D), lambda i: (i, 0)),
            pl.BlockSpec((_BR, 16), lambda i: (i, 0)),
            pl.BlockSpec((1, D), lambda i: (0, 0)),
            pl.BlockSpec((D, D), lambda i: (0, 0)),
        ],
        out_specs=pl.BlockSpec((_BR, D), lambda i: (i, 0)),
        out_shape=jax.ShapeDtypeStruct((NPAD, D), jnp.float32),
    )(p, y1, dis, b1, W2)


def _final_body(q_ref, y2_ref, dis_ref, b2_ref, out_ref):
    agg = q_ref[0] + q_ref[1] + y2_ref[...]
    out_ref[...] = jnp.maximum(agg * dis_ref[:, :1] + b2_ref[...], 0.0)


def _final_layer(q, y2, dis, b2):
    return pl.pallas_call(
        _final_body,
        grid=(_GRID,),
        in_specs=[
            pl.BlockSpec((NC, _BR, D), lambda i: (0, i, 0)),
            pl.BlockSpec((_BR, D), lambda i: (i, 0)),
            pl.BlockSpec((_BR, 16), lambda i: (i, 0)),
            pl.BlockSpec((1, D), lambda i: (0, 0)),
        ],
        out_specs=pl.BlockSpec((_BR, D), lambda i: (i, 0)),
        out_shape=jax.ShapeDtypeStruct((NPAD, D), jnp.float32),
    )(q, y2, dis, b2)


def kernel(x, edge_index, W1, b1, W2, b2):
    npad_e = EPAD - E
    pidx = jnp.arange(npad_e, dtype=jnp.int32)
    pad_trash = N + pidx % (NPAD - N)
    src2d = jnp.concatenate(
        [edge_index[0].astype(jnp.int32), pad_trash]
    ).reshape(NW, NCHUNK, CHUNK)
    dst2d = jnp.concatenate(
        [edge_index[1].astype(jnp.int32), pidx % N]
    ).reshape(NW, NCHUNK, CHUNK)
    dst2d_deg = jnp.concatenate(
        [edge_index[1].astype(jnp.int32), pad_trash]
    ).reshape(NW, NCHUNK, CHUNK)
    xp = jnp.pad(x, ((0, NPAD - N), (0, 0)))
    zeros128 = jnp.zeros((NPAD, D), jnp.float32)
    b1r = b1.reshape(1, D)
    b2r = b2.reshape(1, D)

    degp = _deg_kernel()(dst2d_deg, zeros128)
    xw1 = _matmul(xp, W1)
    y1, dis = _first_layer(xw1, degp)
    p = _agg_kernel()(y1, src2d, dst2d, zeros128)
    y2 = _mid_layer(p, y1, dis, b1r, W2)
    q = _agg_kernel()(y2, src2d, dst2d, zeros128)
    return _final_layer(q, y2, dis, b2r)[:N]


# speedup vs baseline: 3.4191x; 1.3645x over previous
import functools

import jax
import jax.numpy as jnp
from jax import lax
from jax.experimental import pallas as pl
from jax.experimental.pallas import tpu as pltpu
from jax.experimental.pallas import tpu_sc as plsc

N = 10000
E = 320000
D = 128

NC = 2
NS = 16
NW = NC * NS
EPW = E // NW
CHUNK = 128
NCHUNK = 80
HALF = NCHUNK // 2
EPAD = NW * NCHUNK * CHUNK
NPAD = 10112
RPT = NPAD // NS

@functools.cache
def _mesh():
    return plsc.VectorSubcoreMesh(
        core_axis_name="c", subcore_axis_name="s", num_cores=NC, num_subcores=NS
    )


def _deg_body(dst2d, zeros_hbm, out, idx_v, ones_v, acc):
    c = lax.axis_index("c")
    s = lax.axis_index("s")
    wid = c * NS + s
    pltpu.sync_copy(dst2d.at[wid], idx_v)
    ones = jnp.ones((16,), jnp.float32)

    def obody(i, carry):
        r = i // (D // 16)
        k = i % (D // 16)
        ones_v[r, pl.ds(k * 16, 16)] = ones
        return carry

    lax.fori_loop(0, CHUNK * (D // 16), obody, 0)
    pltpu.sync_copy(zeros_hbm.at[pl.ds(s * RPT, RPT)], acc.at[pl.ds(s * RPT, RPT)])
    plsc.subcore_barrier()

    def body(j, carry):
        pltpu.sync_copy(ones_v, acc.at[idx_v.at[j]], add=True)
        return carry

    lax.fori_loop(0, NCHUNK, body, 0)
    plsc.subcore_barrier()
    pltpu.sync_copy(acc.at[pl.ds(s * RPT, RPT)], out.at[c, pl.ds(s * RPT, RPT)])


@functools.cache
def _deg_kernel():
    return pl.kernel(
        _deg_body,
        out_type=jax.ShapeDtypeStruct((NC, NPAD, D), jnp.float32),
        mesh=_mesh(),
        scratch_types=[
            pltpu.VMEM((NCHUNK, CHUNK), jnp.int32),
            pltpu.VMEM((CHUNK, D), jnp.float32),
            pltpu.VMEM_SHARED((NPAD, D), jnp.float32),
        ],
    )


def _agg_body(y, src2d, dst2d, zeros_hbm, out,
              sidx_v, didx_v, rows0, rows1, sem0, sem1, acc):
    c = lax.axis_index("c")
    s = lax.axis_index("s")
    wid = c * NS + s
    pltpu.sync_copy(zeros_hbm.at[pl.ds(s * RPT, RPT)], acc.at[pl.ds(s * RPT, RPT)])
    plsc.subcore_barrier()

    for h in range(2):
        pltpu.sync_copy(src2d.at[wid, pl.ds(h * HALF, HALF)], sidx_v)
        pltpu.sync_copy(dst2d.at[wid, pl.ds(h * HALF, HALF)], didx_v)
        pltpu.async_copy(y.at[sidx_v.at[0]], rows0, sem0)
        pltpu.async_copy(y.at[sidx_v.at[1]], rows1, sem1)

        def body(i, carry):
            j0 = 2 * i
            pltpu.make_async_copy(y.at[sidx_v.at[j0]], rows0, sem0).wait()
            pltpu.sync_copy(rows0, acc.at[didx_v.at[j0]], add=True)

            @pl.when(j0 + 2 < HALF)
            def _():
                pltpu.async_copy(y.at[sidx_v.at[j0 + 2]], rows0, sem0)

            j1 = j0 + 1
            pltpu.make_async_copy(y.at[sidx_v.at[j1]], rows1, sem1).wait()
            pltpu.sync_copy(rows1, acc.at[didx_v.at[j1]], add=True)

            @pl.when(j1 + 2 < HALF)
            def _():
                pltpu.async_copy(y.at[sidx_v.at[j1 + 2]], rows1, sem1)

            return carry

        lax.fori_loop(0, HALF // 2, body, 0)
    plsc.subcore_barrier()
    pltpu.sync_copy(acc.at[pl.ds(s * RPT, RPT)], out.at[c, pl.ds(s * RPT, RPT)])


@functools.cache
def _agg_kernel():
    return pl.kernel(
        _agg_body,
        out_type=jax.ShapeDtypeStruct((NC, NPAD, D), jnp.float32),
        mesh=_mesh(),
        scratch_types=[
            pltpu.VMEM((HALF, CHUNK), jnp.int32),
            pltpu.VMEM((HALF, CHUNK), jnp.int32),
            pltpu.VMEM((CHUNK, D), jnp.float32),
            pltpu.VMEM((CHUNK, D), jnp.float32),
            pltpu.SemaphoreType.DMA,
            pltpu.SemaphoreType.DMA,
            pltpu.VMEM_SHARED((NPAD, D), jnp.float32),
        ],
    )


_BR = 2528
_GRID = NPAD // _BR


def _matmul_body(x_ref, w_ref, xw_ref):
    xw_ref[...] = jnp.dot(x_ref[...], w_ref[...],
                          preferred_element_type=jnp.float32)


def _matmul(x, W):
    return pl.pallas_call(
        _matmul_body,
        grid=(_GRID,),
        in_specs=[
            pl.BlockSpec((_BR, D), lambda i: (i, 0)),
            pl.BlockSpec((D, D), lambda i: (0, 0)),
        ],
        out_specs=pl.BlockSpec((_BR, D), lambda i: (i, 0)),
        out_shape=jax.ShapeDtypeStruct((NPAD, D), jnp.float32),
    )(x, W)


def _scale_body(xw_ref, degp_ref, y_ref, dis_ref):
    deg = degp_ref[0, :, :1] + degp_ref[1, :, :1] + 1.0
    dis = lax.rsqrt(deg)
    y_ref[...] = xw_ref[...] * dis
    dis_ref[...] = jnp.broadcast_to(dis, (_BR, 16))


def _first_layer(xw, degp):
    return pl.pallas_call(
        _scale_body,
        grid=(_GRID,),
        in_specs=[
            pl.BlockSpec((_BR, D), lambda i: (i, 0)),
            pl.BlockSpec((NC, _BR, D), lambda i: (0, i, 0)),
        ],
        out_specs=[
            pl.BlockSpec((_BR, D), lambda i: (i, 0)),
            pl.BlockSpec((_BR, 16), lambda i: (i, 0)),
        ],
        out_shape=[
            jax.ShapeDtypeStruct((NPAD, D), jnp.float32),
            jax.ShapeDtypeStruct((NPAD, 16), jnp.float32),
        ],
    )(xw, degp)


def _mid_body(p_ref, y1_ref, dis_ref, b1_ref, w2_ref, y2_ref):
    agg = p_ref[0] + p_ref[1] + y1_ref[...]
    dcol = dis_ref[:, :1]
    h = jnp.maximum(agg * dcol + b1_ref[...], 0.0)
    y2 = jnp.dot(h, w2_ref[...], preferred_element_type=jnp.float32) * dcol
    row = pl.program_id(0) * _BR + lax.broadcasted_iota(jnp.int32, (_BR, 1), 0)
    y2_ref[...] = jnp.where(row < N, y2, 0.0)


def _mid_layer(p, y1, dis, b1, W2):
    return pl.pallas_call(
        _mid_body,
        grid=(_GRID,),
        in_specs=[
            pl.BlockSpec((NC, _BR, D), lambda i: (0, i, 0)),
            pl.BlockSpec((_BR, D), lambda i: (i, 0)),
            pl.BlockSpec((_BR, 16), lambda i: (i, 0)),
            pl.BlockSpec((1, D), lambda i: (0, 0)),
            pl.BlockSpec((D, D), lambda i: (0, 0)),
        ],
        out_specs=pl.BlockSpec((_BR, D), lambda i: (i, 0)),
        out_shape=jax.ShapeDtypeStruct((NPAD, D), jnp.float32),
    )(p, y1, dis, b1, W2)


def _final_body(q_ref, y2_ref, dis_ref, b2_ref, out_ref):
    agg = q_ref[0] + q_ref[1] + y2_ref[...]
    out_ref[...] = jnp.maximum(agg * dis_ref[:, :1] + b2_ref[...], 0.0)


def _final_layer(q, y2, dis, b2):
    return pl.pallas_call(
        _final_body,
        grid=(_GRID,),
        in_specs=[
            pl.BlockSpec((NC, _BR, D), lambda i: (0, i, 0)),
            pl.BlockSpec((_BR, D), lambda i: (i, 0)),
            pl.BlockSpec((_BR, 16), lambda i: (i, 0)),
            pl.BlockSpec((1, D), lambda i: (0, 0)),
        ],
        out_specs=pl.BlockSpec((_BR, D), lambda i: (i, 0)),
        out_shape=jax.ShapeDtypeStruct((NPAD, D), jnp.float32),
    )(q, y2, dis, b2)


def kernel(x, edge_index, W1, b1, W2, b2):
    npad_e = EPAD - E
    pidx = jnp.arange(npad_e, dtype=jnp.int32)
    pad_trash = N + pidx % (NPAD - N)
    src2d = jnp.concatenate(
        [edge_index[0].astype(jnp.int32), pad_trash]
    ).reshape(NW, NCHUNK, CHUNK)
    dst2d = jnp.concatenate(
        [edge_index[1].astype(jnp.int32), pidx % N]
    ).reshape(NW, NCHUNK, CHUNK)
    dst2d_deg = jnp.concatenate(
        [edge_index[1].astype(jnp.int32), pad_trash]
    ).reshape(NW, NCHUNK, CHUNK)
    xp = jnp.pad(x, ((0, NPAD - N), (0, 0)))
    zeros128 = jnp.zeros((NPAD, D), jnp.float32)
    b1r = b1.reshape(1, D)
    b2r = b2.reshape(1, D)

    degp = _deg_kernel()(dst2d_deg, zeros128)
    xw1 = _matmul(xp, W1)
    y1, dis = _first_layer(xw1, degp)
    p = _agg_kernel()(y1, src2d, dst2d, zeros128)
    y2 = _mid_layer(p, y1, dis, b1r, W2)
    q = _agg_kernel()(y2, src2d, dst2d, zeros128)
    return _final_layer(q, y2, dis, b2r)[:N]
